# baseline (speedup 1.0000x reference)
import functools

import jax
import jax.numpy as jnp
from jax import lax
from jax.experimental import pallas as pl
from jax.experimental.pallas import tpu as pltpu
from jax.experimental.pallas import tpu_sc as plsc

B = 32768
D = 64
NW = 32
ROWS_PER_W = B // NW
CHUNK = 128
N_CHUNKS = ROWS_PER_W // CHUNK
D_PAD = 128
TSTRIDE = CHUNK + 1
V = 100000


def _sc_scores_body(h_idx_hbm, t_idx_hbm, r_idx_hbm, ent_hbm, rel_hbm,
                    out_hbm, idxh_v, idxt_v, idxr_v,
                    h0_v, t0_v, r0_v, h1_v, t1_v, r1_v, tbuf_v,
                    s_v, sem0, sem1):
    wid = lax.axis_index("s") * 2 + lax.axis_index("c")
    lane = lax.iota(jnp.int32, 16)
    lane_c = lane * TSTRIDE
    base = wid * ROWS_PER_W

    pltpu.sync_copy(h_idx_hbm.at[pl.ds(base, ROWS_PER_W)], idxh_v)
    pltpu.sync_copy(t_idx_hbm.at[pl.ds(base, ROWS_PER_W)], idxt_v)
    pltpu.sync_copy(r_idx_hbm.at[pl.ds(base, ROWS_PER_W)], idxr_v)

    bufs = ((h0_v, t0_v, r0_v), (h1_v, t1_v, r1_v))
    sems = (sem0, sem1)

    def fire(c, slot):
        hb, tb, rb = bufs[slot]
        sl = pl.ds(c * CHUNK, CHUNK)
        return (
            pltpu.async_copy(ent_hbm.at[idxh_v.at[sl]], hb, sems[slot]),
            pltpu.async_copy(ent_hbm.at[idxt_v.at[sl]], tb, sems[slot]),
            pltpu.async_copy(rel_hbm.at[idxr_v.at[sl]], rb, sems[slot]),
        )

    def compute(c, slot):
        hb, tb, rb = bufs[slot]

        def row_body(row, _):
            q = jnp.zeros((16,), jnp.float32)
            for j in range(D // 16):
                hj = hb[row, pl.ds(j * 16, 16)]
                tj = tb[row, pl.ds(j * 16, 16)]
                rj = rb[row, pl.ds(j * 16, 16)]
                q = q + hj * rj * tj
            plsc.store_scatter(tbuf_v, [lane_c + row], q)
            return 0

        lax.fori_loop(0, CHUNK, row_body, 0, unroll=4)

        def group_body(g, _):
            gbase = pl.multiple_of(g * 16, 16)
            acc = tbuf_v[pl.ds(gbase, 16)]
            for l in range(1, 16):
                acc = acc + tbuf_v[pl.ds(gbase + l * TSTRIDE, 16)]
            s_v[pl.ds(c * CHUNK + gbase, 16)] = -acc
            return 0

        lax.fori_loop(0, CHUNK // 16, group_body, 0)

    pending = fire(0, 0)
    for c in range(N_CHUNKS):
        nxt = fire(c + 1, (c + 1) % 2) if c + 1 < N_CHUNKS else None
        for cp in pending:
            cp.wait()
        compute(c, c % 2)
        pending = nxt

    pltpu.sync_copy(s_v, out_hbm.at[pl.ds(base, ROWS_PER_W)])


_sc_scores = functools.partial(
    pl.kernel,
    mesh=plsc.VectorSubcoreMesh(core_axis_name="c", subcore_axis_name="s"),
    out_type=jax.ShapeDtypeStruct((B,), jnp.float32),
    compiler_params=pltpu.CompilerParams(
        needs_layout_passes=False, use_tc_tiling_on_sc=False),
    scratch_types=[
        pltpu.VMEM((ROWS_PER_W,), jnp.int32),
        pltpu.VMEM((ROWS_PER_W,), jnp.int32),
        pltpu.VMEM((ROWS_PER_W,), jnp.int32),
        pltpu.VMEM((CHUNK, D_PAD), jnp.float32),
        pltpu.VMEM((CHUNK, D_PAD), jnp.float32),
        pltpu.VMEM((CHUNK, D_PAD), jnp.float32),
        pltpu.VMEM((CHUNK, D_PAD), jnp.float32),
        pltpu.VMEM((CHUNK, D_PAD), jnp.float32),
        pltpu.VMEM((CHUNK, D_PAD), jnp.float32),
        pltpu.VMEM((16 * TSTRIDE,), jnp.float32),
        pltpu.VMEM((ROWS_PER_W,), jnp.float32),
        pltpu.SemaphoreType.DMA,
        pltpu.SemaphoreType.DMA,
    ],
)(_sc_scores_body)


def _tct_body(x_ref, o_ref):
    x = x_ref[...]
    o_ref[...] = jnp.concatenate(
        [x.T, jnp.zeros((128, D_PAD - D), jnp.float32)], axis=1)


_tc_transpose = pl.pallas_call(
    _tct_body,
    grid=( (V + 127) // 128, ),
    in_specs=[pl.BlockSpec((D, 128), lambda i: (0, i))],
    out_specs=pl.BlockSpec((128, D_PAD), lambda i: (i, 0)),
    out_shape=jax.ShapeDtypeStruct((V, D_PAD), jnp.float32),
)


def _loss_body(x_ref, o_ref):
    x = x_ref[...]
    p = x[:128, :]
    n = x[128:, :]

    def softplus(v):
        return jnp.maximum(v, 0.0) + jnp.log1p(jnp.exp(-jnp.abs(v)))

    tot = jnp.sum(softplus(-p)) + jnp.sum(softplus(n))
    o_ref[0] = tot / (2.0 * (B // 2))


_loss = pl.pallas_call(
    _loss_body,
    out_shape=jax.ShapeDtypeStruct((1,), jnp.float32),
    in_specs=[pl.BlockSpec(memory_space=pltpu.VMEM)],
    out_specs=pl.BlockSpec(memory_space=pltpu.SMEM),
)


def kernel(data, ent_emb, rel_emb):
    ent_pad = jnp.pad(ent_emb, ((0, 0), (0, D_PAD - D)))
    rel_pad = _tc_transpose(rel_emb.T)
    score = _sc_scores(data[0], data[1], data[2], ent_pad, rel_pad)
    loss = _loss(score.reshape(B // 128, 128))[0]
    return loss.reshape(())


# speedup vs baseline: 3.3341x; 3.3341x over previous
import functools

import jax
import jax.numpy as jnp
from jax import lax
from jax.experimental import pallas as pl
from jax.experimental.pallas import tpu as pltpu
from jax.experimental.pallas import tpu_sc as plsc

B = 32768
D = 64
NW = 32
ROWS_PER_W = B // NW
CHUNK = 128
N_CHUNKS = ROWS_PER_W // CHUNK
D_PAD = 128
TSTRIDE = CHUNK + 1
V = 100000


def _sc_scores_body(h_idx_hbm, t_idx_hbm, r_idx_hbm, ent_hbm, rel_hbm,
                    out_hbm, idxh_v, idxt_v, idxr_v,
                    h0_v, t0_v, r0_v, h1_v, t1_v, r1_v, tbuf_v,
                    s_v, sem0, sem1):
    wid = lax.axis_index("s") * 2 + lax.axis_index("c")
    lane = lax.iota(jnp.int32, 16)
    lane_c = lane * TSTRIDE
    base = wid * ROWS_PER_W

    pltpu.sync_copy(h_idx_hbm.at[pl.ds(base, ROWS_PER_W)], idxh_v)
    pltpu.sync_copy(t_idx_hbm.at[pl.ds(base, ROWS_PER_W)], idxt_v)
    pltpu.sync_copy(r_idx_hbm.at[pl.ds(base, ROWS_PER_W)], idxr_v)

    bufs = ((h0_v, t0_v, r0_v), (h1_v, t1_v, r1_v))
    sems = (sem0, sem1)

    def fire(c, slot):
        hb, tb, rb = bufs[slot]
        sl = pl.ds(c * CHUNK, CHUNK)
        return (
            pltpu.async_copy(ent_hbm.at[idxh_v.at[sl]], hb, sems[slot]),
            pltpu.async_copy(ent_hbm.at[idxt_v.at[sl]], tb, sems[slot]),
            pltpu.async_copy(rel_hbm.at[idxr_v.at[sl]], rb, sems[slot]),
        )

    def compute(c, slot):
        hb, tb, rb = bufs[slot]

        def row_body(row, _):
            q = jnp.zeros((16,), jnp.float32)
            for j in range(D // 16):
                hj = hb[row, pl.ds(j * 16, 16)]
                tj = tb[row, pl.ds(j * 16, 16)]
                rj = rb[row, pl.ds(j * 16, 16)]
                q = q + hj * rj * tj
            plsc.store_scatter(tbuf_v, [lane_c + row], q)
            return 0

        lax.fori_loop(0, CHUNK, row_body, 0, unroll=4)

        def group_body(g, _):
            gbase = pl.multiple_of(g * 16, 16)
            acc = tbuf_v[pl.ds(gbase, 16)]
            for l in range(1, 16):
                acc = acc + tbuf_v[pl.ds(gbase + l * TSTRIDE, 16)]
            s_v[pl.ds(c * CHUNK + gbase, 16)] = -acc
            return 0

        lax.fori_loop(0, CHUNK // 16, group_body, 0)

    pending = fire(0, 0)
    for c in range(N_CHUNKS):
        nxt = fire(c + 1, (c + 1) % 2) if c + 1 < N_CHUNKS else None
        for cp in pending:
            cp.wait()
        compute(c, c % 2)
        pending = nxt

    pltpu.sync_copy(s_v, out_hbm.at[pl.ds(base, ROWS_PER_W)])


_sc_scores = functools.partial(
    pl.kernel,
    mesh=plsc.VectorSubcoreMesh(core_axis_name="c", subcore_axis_name="s"),
    out_type=jax.ShapeDtypeStruct((B,), jnp.float32),
    compiler_params=pltpu.CompilerParams(
        needs_layout_passes=False, use_tc_tiling_on_sc=False),
    scratch_types=[
        pltpu.VMEM((ROWS_PER_W,), jnp.int32),
        pltpu.VMEM((ROWS_PER_W,), jnp.int32),
        pltpu.VMEM((ROWS_PER_W,), jnp.int32),
        pltpu.VMEM((CHUNK, D_PAD), jnp.float32),
        pltpu.VMEM((CHUNK, D_PAD), jnp.float32),
        pltpu.VMEM((CHUNK, D_PAD), jnp.float32),
        pltpu.VMEM((CHUNK, D_PAD), jnp.float32),
        pltpu.VMEM((CHUNK, D_PAD), jnp.float32),
        pltpu.VMEM((CHUNK, D_PAD), jnp.float32),
        pltpu.VMEM((16 * TSTRIDE,), jnp.float32),
        pltpu.VMEM((ROWS_PER_W,), jnp.float32),
        pltpu.SemaphoreType.DMA,
        pltpu.SemaphoreType.DMA,
    ],
)(_sc_scores_body)


def _tct_body(x_ref, o_ref):
    x = x_ref[...]
    o_ref[...] = jnp.concatenate(
        [x.T, jnp.zeros((128, D_PAD - D), jnp.float32)], axis=1)


_tc_transpose = pl.pallas_call(
    _tct_body,
    grid=( (V + 127) // 128, ),
    in_specs=[pl.BlockSpec((D, 128), lambda i: (0, i))],
    out_specs=pl.BlockSpec((128, D_PAD), lambda i: (i, 0)),
    out_shape=jax.ShapeDtypeStruct((V, D_PAD), jnp.float32),
)


def _loss_body(x_ref, o_ref):
    x = x_ref[...]
    p = x[:128, :]
    n = x[128:, :]

    def softplus(v):
        return jnp.maximum(v, 0.0) + jnp.log1p(jnp.exp(-jnp.abs(v)))

    tot = jnp.sum(softplus(-p)) + jnp.sum(softplus(n))
    o_ref[0] = tot / (2.0 * (B // 2))


_loss = pl.pallas_call(
    _loss_body,
    out_shape=jax.ShapeDtypeStruct((1,), jnp.float32),
    in_specs=[pl.BlockSpec(memory_space=pltpu.VMEM)],
    out_specs=pl.BlockSpec(memory_space=pltpu.SMEM),
)


def kernel(data, ent_emb, rel_emb):
    ent_pad = jnp.pad(ent_emb, ((0, 0), (0, D_PAD - D)))
    rel_pad = jnp.pad(rel_emb, ((0, 0), (0, D_PAD - D)))
    score = _sc_scores(data[0], data[1], data[2], ent_pad, rel_pad)
    loss = _loss(score.reshape(B // 128, 128))[0]
    return loss.reshape(())


# speedup vs baseline: 3.5619x; 1.0683x over previous
import functools

import jax
import jax.numpy as jnp
from jax import lax
from jax.experimental import pallas as pl
from jax.experimental.pallas import tpu as pltpu
from jax.experimental.pallas import tpu_sc as plsc

B = 32768
D = 64
NW = 32
ROWS_PER_W = B // NW
CHUNK = 128
N_CHUNKS = ROWS_PER_W // CHUNK
D_PAD = 128
TSTRIDE = CHUNK + 1
V = 100000


def _sc_scores_body(h_idx_hbm, t_idx_hbm, r_idx_hbm, ent_hbm, rel_hbm,
                    out_hbm, idxh_v, idxt_v, idxr_v,
                    h0_v, t0_v, r0_v, h1_v, t1_v, r1_v, tbuf_v,
                    s_v, sem0, sem1):
    wid = lax.axis_index("s") * 2 + lax.axis_index("c")
    lane = lax.iota(jnp.int32, 16)
    lane_c = lane * TSTRIDE
    base = wid * ROWS_PER_W

    pltpu.sync_copy(h_idx_hbm.at[pl.ds(base, ROWS_PER_W)], idxh_v)
    pltpu.sync_copy(t_idx_hbm.at[pl.ds(base, ROWS_PER_W)], idxt_v)
    pltpu.sync_copy(r_idx_hbm.at[pl.ds(base, ROWS_PER_W)], idxr_v)

    bufs = ((h0_v, t0_v, r0_v), (h1_v, t1_v, r1_v))
    sems = (sem0, sem1)

    def fire(c, slot):
        hb, tb, rb = bufs[slot]
        sl = pl.ds(c * CHUNK, CHUNK)
        return (
            pltpu.async_copy(ent_hbm.at[idxh_v.at[sl]], hb, sems[slot]),
            pltpu.async_copy(ent_hbm.at[idxt_v.at[sl]], tb, sems[slot]),
            pltpu.async_copy(rel_hbm.at[idxr_v.at[sl]], rb, sems[slot]),
        )

    def compute(c, slot):
        hb, tb, rb = bufs[slot]

        def row_body(row, _):
            q = jnp.zeros((16,), jnp.float32)
            for j in range(D // 16):
                hj = hb[row, pl.ds(j * 16, 16)]
                tj = tb[row, pl.ds(j * 16, 16)]
                rj = rb[row, pl.ds(D + j * 16, 16)]
                q = q + hj * rj * tj
            plsc.store_scatter(tbuf_v, [lane_c + row], q)
            return 0

        lax.fori_loop(0, CHUNK, row_body, 0, unroll=4)

        def group_body(g, _):
            gbase = pl.multiple_of(g * 16, 16)
            acc = tbuf_v[pl.ds(gbase, 16)]
            for l in range(1, 16):
                acc = acc + tbuf_v[pl.ds(gbase + l * TSTRIDE, 16)]
            s_v[pl.ds(c * CHUNK + gbase, 16)] = -acc
            return 0

        lax.fori_loop(0, CHUNK // 16, group_body, 0)

    pending = fire(0, 0)
    for c in range(N_CHUNKS):
        nxt = fire(c + 1, (c + 1) % 2) if c + 1 < N_CHUNKS else None
        for cp in pending:
            cp.wait()
        compute(c, c % 2)
        pending = nxt

    pltpu.sync_copy(s_v, out_hbm.at[pl.ds(base, ROWS_PER_W)])


_sc_scores = functools.partial(
    pl.kernel,
    mesh=plsc.VectorSubcoreMesh(core_axis_name="c", subcore_axis_name="s"),
    out_type=jax.ShapeDtypeStruct((B,), jnp.float32),
    compiler_params=pltpu.CompilerParams(
        needs_layout_passes=False, use_tc_tiling_on_sc=False),
    scratch_types=[
        pltpu.VMEM((ROWS_PER_W,), jnp.int32),
        pltpu.VMEM((ROWS_PER_W,), jnp.int32),
        pltpu.VMEM((ROWS_PER_W,), jnp.int32),
        pltpu.VMEM((CHUNK, D_PAD), jnp.float32),
        pltpu.VMEM((CHUNK, D_PAD), jnp.float32),
        pltpu.VMEM((CHUNK, D_PAD), jnp.float32),
        pltpu.VMEM((CHUNK, D_PAD), jnp.float32),
        pltpu.VMEM((CHUNK, D_PAD), jnp.float32),
        pltpu.VMEM((CHUNK, D_PAD), jnp.float32),
        pltpu.VMEM((16 * TSTRIDE,), jnp.float32),
        pltpu.VMEM((ROWS_PER_W,), jnp.float32),
        pltpu.SemaphoreType.DMA,
        pltpu.SemaphoreType.DMA,
    ],
)(_sc_scores_body)


def _tct_body(x_ref, o_ref):
    x = x_ref[...]
    o_ref[...] = jnp.concatenate(
        [x.T, jnp.zeros((128, D_PAD - D), jnp.float32)], axis=1)


_tc_transpose = pl.pallas_call(
    _tct_body,
    grid=( (V + 127) // 128, ),
    in_specs=[pl.BlockSpec((D, 128), lambda i: (0, i))],
    out_specs=pl.BlockSpec((128, D_PAD), lambda i: (i, 0)),
    out_shape=jax.ShapeDtypeStruct((V, D_PAD), jnp.float32),
)


def _loss_body(x_ref, o_ref):
    x = x_ref[...]
    p = x[:128, :]
    n = x[128:, :]

    def softplus(v):
        return jnp.maximum(v, 0.0) + jnp.log1p(jnp.exp(-jnp.abs(v)))

    tot = jnp.sum(softplus(-p)) + jnp.sum(softplus(n))
    o_ref[0] = tot / (2.0 * (B // 2))


_loss = pl.pallas_call(
    _loss_body,
    out_shape=jax.ShapeDtypeStruct((1,), jnp.float32),
    in_specs=[pl.BlockSpec(memory_space=pltpu.VMEM)],
    out_specs=pl.BlockSpec(memory_space=pltpu.SMEM),
)


def kernel(data, ent_emb, rel_emb):
    both = jnp.concatenate([ent_emb, rel_emb], axis=1)
    score = _sc_scores(data[0], data[1], data[2], both, both)
    loss = _loss(score.reshape(B // 128, 128))[0]
    return loss.reshape(())
